# Initial kernel scaffold; baseline (speedup 1.0000x reference)
import jax
import jax.numpy as jnp
from jax.experimental import pallas as pl


def kernel(z, codebook):
    raise NotImplementedError("write your pallas kernel here")


# speedup vs baseline: 1.0018x; 1.0018x over previous
import jax
import jax.numpy as jnp
from jax.experimental import pallas as pl
from jax.experimental.pallas import tpu as pltpu

NE = 8192
ED = 32
TOK = 8192
TILE = 256
CHUNK = 2048
CC = 0.25


def _bf16(x):
    return x.astype(jnp.bfloat16).astype(jnp.float32)


def _vq_body(zb_ref, z2_ref, cb_ref, c2_ref, idx_ref, q_ref, loss_ref):
    m = jax.lax.dot_general(
        zb_ref[...], cb_ref[...], (((1,), (1,)), ((), ())),
        preferred_element_type=jnp.float32)
    dist = (z2_ref[...] + c2_ref[...]) - m
    lanes = jax.lax.broadcasted_iota(jnp.int32, dist.shape, 1)
    v = None
    for g in range(NE // CHUNK):
        blk = dist[:, g * CHUNK:(g + 1) * CHUNK]
        mg = jnp.min(blk, axis=1, keepdims=True)
        lg = lanes[:, g * CHUNK:(g + 1) * CHUNK]
        ig = jnp.min(jnp.where(blk == mg, lg, jnp.int32(NE)),
                     axis=1, keepdims=True)
        if v is None:
            v, ix, wd = _bf16(mg), ig, mg
        else:
            upd = mg < v
            v = jnp.where(upd, _bf16(mg), v)
            ix = jnp.where(upd, ig, ix)
            wd = jnp.where(upd, mg, wd)
    idx_ref[...] = ix
    enc = jnp.where(lanes == ix, 1.0, 0.0).astype(jnp.bfloat16)
    q_ref[...] = jax.lax.dot_general(
        enc, cb_ref[...], (((1,), (0,)), ((), ())),
        preferred_element_type=jnp.float32)

    @pl.when(pl.program_id(0) == 0)
    def _init():
        loss_ref[0, 0] = 0.0

    loss_ref[0, 0] += jnp.sum(wd)


def _vq_call(zb, z2, cb, c2):
    grid = TOK // TILE
    return pl.pallas_call(
        _vq_body,
        grid=(grid,),
        in_specs=[
            pl.BlockSpec((TILE, ED), lambda i: (i, 0)),
            pl.BlockSpec((TILE, 1), lambda i: (i, 0)),
            pl.BlockSpec((NE, ED), lambda i: (0, 0)),
            pl.BlockSpec((1, NE), lambda i: (0, 0)),
        ],
        out_specs=[
            pl.BlockSpec((TILE, 1), lambda i: (i, 0)),
            pl.BlockSpec((TILE, ED), lambda i: (i, 0)),
            pl.BlockSpec(memory_space=pltpu.SMEM),
        ],
        out_shape=[
            jax.ShapeDtypeStruct((TOK, 1), jnp.int32),
            jax.ShapeDtypeStruct((TOK, ED), jnp.float32),
            jax.ShapeDtypeStruct((1, 1), jnp.float32),
        ],
    )(zb, z2, cb, c2)


def kernel(z, codebook):
    zp = jnp.transpose(z, (0, 2, 1))
    z_flat = zp.reshape(-1, ED)
    z2 = jnp.sum(z_flat ** 2, axis=1, keepdims=True)
    c2 = jnp.sum(codebook ** 2, axis=1).reshape(1, NE)
    zb = (2.0 * z_flat).astype(jnp.bfloat16)
    cb = codebook.astype(jnp.bfloat16)
    idx2d, q_flat, loss_sum = _vq_call(zb, z2, cb, c2)
    vq_loss = loss_sum[0, 0] * ((1.0 + CC) / (TOK * ED))
    q_st = z_flat + (q_flat - z_flat)
    q_out = jnp.transpose(q_st.reshape(zp.shape), (0, 2, 1))
    idx_out = idx2d.reshape(z.shape[0], -1)
    return (vq_loss, q_out, idx_out)
